# baseline (speedup 1.0000x reference)
import dataclasses
import functools

import jax
import jax.numpy as jnp
from jax import lax
from jax.experimental import pallas as pl
from jax.experimental.pallas import tpu as pltpu
from jax.experimental.pallas import tpu_sc as plsc

N_NODES = 10000
N_ACC = 10016
D_FEAT = 128
DIM_H = 128
DIM_EXT = 144
N_EDGES = 320000
CHUNK = 112
EDGES_PER_TILE = 10080
E_PAD = 32 * EDGES_PER_TILE
NCHUNKS = EDGES_PER_TILE // CHUNK
ROWS_PER_SUB = N_ACC // 16


def _dot(a, b):
    return lax.dot_general(a, b, (((1,), (0,)), ((), ())),
                           preferred_element_type=jnp.float32)


def _pre_body(x_ref, ws_ref, wd_ref, atts_ref, attd_ref, xe_ref, ad_ref):
    x = x_ref[...]
    xs = _dot(x, ws_ref[...])
    asrc16 = _dot(xs, atts_ref[...])
    xe_ref[...] = jnp.concatenate([xs, asrc16], axis=1)
    xd = _dot(x, wd_ref[...])
    ad_ref[...] = _dot(xd, attd_ref[...])


def _tc_pre(x, W_src, W_dst, atts16, attd16):
    blk = 2000
    grid = (N_NODES // blk,)
    return pl.pallas_call(
        _pre_body,
        grid=grid,
        in_specs=[
            pl.BlockSpec((blk, D_FEAT), lambda i: (i, 0)),
            pl.BlockSpec((D_FEAT, DIM_H), lambda i: (0, 0)),
            pl.BlockSpec((D_FEAT, DIM_H), lambda i: (0, 0)),
            pl.BlockSpec((DIM_H, 16), lambda i: (0, 0)),
            pl.BlockSpec((DIM_H, 16), lambda i: (0, 0)),
        ],
        out_specs=[
            pl.BlockSpec((blk, DIM_EXT), lambda i: (i, 0)),
            pl.BlockSpec((blk, 16), lambda i: (i, 0)),
        ],
        out_shape=[
            jax.ShapeDtypeStruct((N_NODES, DIM_EXT), jnp.float32),
            jax.ShapeDtypeStruct((N_NODES, 16), jnp.float32),
        ],
    )(x, W_src, W_dst, atts16, attd16)


def _sc_body(xe_hbm, ad2_hbm, ei_hbm, feat_hbm, den_hbm,
             rows0, rows1, arows0, arows1, sidx0, sidx1, didx0, didx1,
             dsc0, dsc1, p_v, out_sh,
             gsem0, gsem1, ssem0, ssem1, isem0, isem1):
    c = lax.axis_index("c")
    s = lax.axis_index("s")
    wid = c * 16 + s
    edge0 = wid * EDGES_PER_TILE

    rows = (rows0, rows1)
    arows = (arows0, arows1)
    sidx = (sidx0, sidx1)
    didx = (didx0, didx1)
    dsc = (dsc0, dsc1)
    gsem = (gsem0, gsem1)
    ssem = (ssem0, ssem1)
    isem = (isem0, isem1)

    zf = jnp.zeros((16,), jnp.float32)
    lane = lax.iota(jnp.int32, 16)
    c128 = jnp.full((16,), DIM_H, jnp.int32)
    c0 = jnp.zeros((16,), jnp.int32)

    @pl.loop(0, CHUNK)
    def _zero(r):
        for k in range(DIM_EXT // 16):
            rows0[r, pl.ds(k * 16, 16)] = zf

    base_row = s * ROWS_PER_SUB
    for t in range(5):
        pltpu.sync_copy(rows0, out_sh.at[pl.ds(base_row + t * CHUNK, CHUNK)])
    pltpu.sync_copy(rows0.at[pl.ds(0, 66)],
                    out_sh.at[pl.ds(base_row + 5 * CHUNK, 66)])
    plsc.subcore_barrier()

    def issue_idx(i, b, sync=False):
        off = edge0 + i * CHUNK
        if sync:
            pltpu.sync_copy(ei_hbm.at[0, pl.ds(off, CHUNK)], sidx[b])
            pltpu.sync_copy(ei_hbm.at[1, pl.ds(off, CHUNK)], didx[b])
        else:
            pltpu.async_copy(ei_hbm.at[0, pl.ds(off, CHUNK)], sidx[b], isem[b])
            pltpu.async_copy(ei_hbm.at[1, pl.ds(off, CHUNK)], didx[b], isem[b])

    def wait_idx(i, b):
        off = edge0 + i * CHUNK
        pltpu.make_async_copy(ei_hbm.at[0, pl.ds(off, CHUNK)], sidx[b], isem[b]).wait()
        pltpu.make_async_copy(ei_hbm.at[1, pl.ds(off, CHUNK)], didx[b], isem[b]).wait()

    def issue_gather(b):
        pltpu.async_copy(xe_hbm.at[sidx[b]], rows[b], gsem[b])
        pltpu.async_copy(ad2_hbm.at[didx[b]], arows[b], gsem[b])

    def wait_gather(b):
        pltpu.make_async_copy(xe_hbm.at[sidx[b]], rows[b], gsem[b]).wait()
        pltpu.make_async_copy(ad2_hbm.at[didx[b]], arows[b], gsem[b]).wait()

    def issue_scatter(b):
        pltpu.async_copy(rows[b], out_sh.at[dsc[b]], ssem[b], add=True)

    def wait_scatter(b):
        pltpu.make_async_copy(rows[b], out_sh.at[dsc[b]], ssem[b]).wait()

    def compute(b):
        rb = rows[b]
        ab = arows[b]
        for j in range(CHUNK // 16):
            ridx = lane + j * 16
            a = plsc.load_gather(rb, [ridx, c128]) + plsc.load_gather(ab, [ridx, c0])
            e = jnp.where(a >= 0.0, a, 0.2 * a)
            p = jnp.exp(e)
            p_v[pl.ds(j * 16, 16)] = p
            plsc.store_scatter(rb, [ridx, c128], p)

        @pl.loop(0, CHUNK, step=8)
        def _scale(r0):
            for rr in range(8):
                pb = plsc.load_gather(p_v, [jnp.full((16,), r0 + rr, jnp.int32)])
                for k in range(DIM_H // 16):
                    rb[r0 + rr, pl.ds(k * 16, 16)] = rb[r0 + rr, pl.ds(k * 16, 16)] * pb

    def half(i, b):
        b1 = 1 - b
        wait_idx(i + 1, b1)

        @pl.when(i >= 1)
        def _():
            wait_scatter(b1)

        issue_gather(b1)
        wait_gather(b)
        for k in range(CHUNK // 16):
            dsc[b][pl.ds(k * 16, 16)] = didx[b][pl.ds(k * 16, 16)]

        @pl.when(i + 2 < NCHUNKS)
        def _():
            issue_idx(i + 2, b)

        compute(b)
        issue_scatter(b)

    issue_idx(0, 0, sync=True)
    issue_gather(0)
    issue_idx(1, 1)

    @pl.loop(0, NCHUNKS - 2, step=2)
    def _main(i):
        half(i, 0)
        half(i + 1, 1)

    half(NCHUNKS - 2, 0)
    wait_gather(1)
    for k in range(CHUNK // 16):
        dsc1[pl.ds(k * 16, 16)] = didx1[pl.ds(k * 16, 16)]
    compute(1)
    issue_scatter(1)
    wait_scatter(0)
    wait_scatter(1)

    plsc.subcore_barrier()
    out_base = c * N_NODES + base_row

    @pl.when(s < 15)
    def _():
        pltpu.sync_copy(out_sh.at[pl.ds(base_row, ROWS_PER_SUB), pl.ds(0, DIM_H)],
                        feat_hbm.at[pl.ds(out_base, ROWS_PER_SUB)])
        pltpu.sync_copy(out_sh.at[pl.ds(base_row, ROWS_PER_SUB), pl.ds(DIM_H, 16)],
                        den_hbm.at[pl.ds(out_base, ROWS_PER_SUB)])

    @pl.when(s == 15)
    def _():
        last = N_NODES - 15 * ROWS_PER_SUB
        pltpu.sync_copy(out_sh.at[pl.ds(base_row, last), pl.ds(0, DIM_H)],
                        feat_hbm.at[pl.ds(out_base, last)])
        pltpu.sync_copy(out_sh.at[pl.ds(base_row, last), pl.ds(DIM_H, 16)],
                        den_hbm.at[pl.ds(out_base, last)])


def _sc_gat(xe, ad2, ei):
    mesh = plsc.VectorSubcoreMesh(core_axis_name="c", subcore_axis_name="s")
    cp = pltpu.CompilerParams(use_tc_tiling_on_sc=False)
    if "needs_layout_passes" in pltpu.CompilerParams.__dataclass_fields__:
        cp = dataclasses.replace(cp, needs_layout_passes=False)
    kern = pl.kernel(
        _sc_body,
        out_type=[
            jax.ShapeDtypeStruct((2 * N_NODES, DIM_H), jnp.float32),
            jax.ShapeDtypeStruct((2 * N_NODES, 16), jnp.float32),
        ],
        mesh=mesh,
        scratch_types=[
            pltpu.VMEM((CHUNK, DIM_EXT), jnp.float32),
            pltpu.VMEM((CHUNK, DIM_EXT), jnp.float32),
            pltpu.VMEM((CHUNK, 16), jnp.float32),
            pltpu.VMEM((CHUNK, 16), jnp.float32),
            pltpu.VMEM((CHUNK,), jnp.int32),
            pltpu.VMEM((CHUNK,), jnp.int32),
            pltpu.VMEM((CHUNK,), jnp.int32),
            pltpu.VMEM((CHUNK,), jnp.int32),
            pltpu.VMEM((CHUNK,), jnp.int32),
            pltpu.VMEM((CHUNK,), jnp.int32),
            pltpu.VMEM((CHUNK,), jnp.float32),
            pltpu.VMEM_SHARED((N_ACC, DIM_EXT), jnp.float32),
            pltpu.SemaphoreType.DMA,
            pltpu.SemaphoreType.DMA,
            pltpu.SemaphoreType.DMA,
            pltpu.SemaphoreType.DMA,
            pltpu.SemaphoreType.DMA,
            pltpu.SemaphoreType.DMA,
        ],
        compiler_params=cp,
    )
    return kern(xe, ad2, ei)


def _post_body(f0_ref, f1_ref, d0_ref, d1_ref, bias_ref, wl_ref, bl_ref,
               o_ref):
    acc = f0_ref[...] + f1_ref[...]
    den = jnp.sum(d0_ref[...] + d1_ref[...], axis=1, keepdims=True)
    h = jnp.maximum(acc / (den + 1e-16) + bias_ref[...], 0.0)
    o_ref[...] = _dot(h, wl_ref[...]) + bl_ref[...]


def _tc_post(feat, den, bias, W_lin, b_lin):
    blk = 2000
    grid = (N_NODES // blk,)
    half_off = N_NODES // blk
    return pl.pallas_call(
        _post_body,
        grid=grid,
        in_specs=[
            pl.BlockSpec((blk, DIM_H), lambda i: (i, 0)),
            pl.BlockSpec((blk, DIM_H), lambda i, o=half_off: (i + o, 0)),
            pl.BlockSpec((blk, 16), lambda i: (i, 0)),
            pl.BlockSpec((blk, 16), lambda i, o=half_off: (i + o, 0)),
            pl.BlockSpec((1, DIM_H), lambda i: (0, 0)),
            pl.BlockSpec((DIM_H, DIM_H), lambda i: (0, 0)),
            pl.BlockSpec((1, DIM_H), lambda i: (0, 0)),
        ],
        out_specs=pl.BlockSpec((blk, DIM_H), lambda i: (i, 0)),
        out_shape=jax.ShapeDtypeStruct((N_NODES, DIM_H), jnp.float32),
    )(feat, feat, den, den, bias, W_lin, b_lin)


def kernel(x, edge_index, W_src, W_dst, att_src, att_dst, bias_conv,
           W_lin, b_lin):
    ei = edge_index.astype(jnp.int32)
    pad = jnp.stack([jnp.zeros((E_PAD - N_EDGES,), jnp.int32),
                     jnp.full((E_PAD - N_EDGES,), N_NODES, jnp.int32)])
    ei = jnp.concatenate([ei, pad], axis=1)

    atts16 = jnp.zeros((DIM_H, 16), jnp.float32).at[:, 0].set(att_src)
    attd16 = jnp.zeros((DIM_H, 16), jnp.float32).at[:, 0].set(att_dst)
    xe, ad2 = _tc_pre(x, W_src, W_dst, atts16, attd16)

    ad2 = jnp.pad(ad2, ((0, N_ACC - N_NODES), (0, 0)))
    feat, den = _sc_gat(xe, ad2, ei)

    return _tc_post(feat, den, bias_conv.reshape(1, DIM_H),
                    W_lin, b_lin.reshape(1, DIM_H))


# speedup vs baseline: 1.3744x; 1.3744x over previous
import dataclasses
import functools

import jax
import jax.numpy as jnp
from jax import lax
from jax.experimental import pallas as pl
from jax.experimental.pallas import tpu as pltpu
from jax.experimental.pallas import tpu_sc as plsc

N_NODES = 10000
D_FEAT = 128
DIM_H = 128
DIM_EXT = 144
N_EDGES = 320000
CHUNK = 80
EDGES_PER_TILE = N_EDGES // 32
NCHUNKS = EDGES_PER_TILE // CHUNK
ROWS_PER_SUB = N_NODES // 16


def _dot(a, b):
    return lax.dot_general(a, b, (((1,), (0,)), ((), ())),
                           preferred_element_type=jnp.float32)


def _pre_body(x_ref, ws_ref, wd_ref, atts_ref, attd_ref, xe_ref, ad_ref):
    x = x_ref[...]
    xs = _dot(x, ws_ref[...])
    asrc16 = _dot(xs, atts_ref[...])
    xe_ref[...] = jnp.concatenate([xs, asrc16], axis=1)
    xd = _dot(x, wd_ref[...])
    ad_ref[...] = _dot(xd, attd_ref[...])


def _tc_pre(x, W_src, W_dst, atts16, attd16):
    blk = 2000
    grid = (N_NODES // blk,)
    return pl.pallas_call(
        _pre_body,
        grid=grid,
        in_specs=[
            pl.BlockSpec((blk, D_FEAT), lambda i: (i, 0)),
            pl.BlockSpec((D_FEAT, DIM_H), lambda i: (0, 0)),
            pl.BlockSpec((D_FEAT, DIM_H), lambda i: (0, 0)),
            pl.BlockSpec((DIM_H, 16), lambda i: (0, 0)),
            pl.BlockSpec((DIM_H, 16), lambda i: (0, 0)),
        ],
        out_specs=[
            pl.BlockSpec((blk, DIM_EXT), lambda i: (i, 0)),
            pl.BlockSpec((blk, 16), lambda i: (i, 0)),
        ],
        out_shape=[
            jax.ShapeDtypeStruct((N_NODES, DIM_EXT), jnp.float32),
            jax.ShapeDtypeStruct((N_NODES, 16), jnp.float32),
        ],
    )(x, W_src, W_dst, atts16, attd16)


def _sc_body(xe_hbm, ad2_hbm, src_hbm, dst_hbm, feat_hbm, den_hbm,
             rows0, rows1, arows0, arows1, sidx0, sidx1, didx0, didx1,
             dsc0, dsc1, p_v, out_sh,
             gsem0, gsem1, ssem0, ssem1, isem0, isem1):
    c = lax.axis_index("c")
    s = lax.axis_index("s")
    wid = c * 16 + s
    edge0 = wid * EDGES_PER_TILE

    rows = (rows0, rows1)
    arows = (arows0, arows1)
    sidx = (sidx0, sidx1)
    didx = (didx0, didx1)
    dsc = (dsc0, dsc1)
    gsem = (gsem0, gsem1)
    ssem = (ssem0, ssem1)
    isem = (isem0, isem1)

    zf = jnp.zeros((16,), jnp.float32)
    lane = lax.iota(jnp.int32, 16)
    c128 = jnp.full((16,), DIM_H, jnp.int32)
    c0 = jnp.zeros((16,), jnp.int32)

    @pl.loop(0, CHUNK)
    def _zero(r):
        for k in range(DIM_EXT // 16):
            rows0[r, pl.ds(k * 16, 16)] = zf

    base_row = s * ROWS_PER_SUB
    for t in range(7):
        pltpu.sync_copy(rows0, out_sh.at[pl.ds(base_row + t * CHUNK, CHUNK)])
    pltpu.sync_copy(rows0.at[pl.ds(0, 65)],
                    out_sh.at[pl.ds(base_row + 7 * CHUNK, 65)])
    plsc.subcore_barrier()

    def issue_idx(i, b, sync=False):
        off = edge0 + i * CHUNK
        if sync:
            pltpu.sync_copy(src_hbm.at[pl.ds(off, CHUNK)], sidx[b])
            pltpu.sync_copy(dst_hbm.at[pl.ds(off, CHUNK)], didx[b])
        else:
            pltpu.async_copy(src_hbm.at[pl.ds(off, CHUNK)], sidx[b], isem[b])
            pltpu.async_copy(dst_hbm.at[pl.ds(off, CHUNK)], didx[b], isem[b])

    def wait_idx(i, b):
        off = edge0 + i * CHUNK
        pltpu.make_async_copy(src_hbm.at[pl.ds(off, CHUNK)], sidx[b], isem[b]).wait()
        pltpu.make_async_copy(dst_hbm.at[pl.ds(off, CHUNK)], didx[b], isem[b]).wait()

    def issue_gather(b):
        pltpu.async_copy(xe_hbm.at[sidx[b]], rows[b], gsem[b])
        pltpu.async_copy(ad2_hbm.at[didx[b]], arows[b], gsem[b])

    def wait_gather(b):
        pltpu.make_async_copy(xe_hbm.at[sidx[b]], rows[b], gsem[b]).wait()
        pltpu.make_async_copy(ad2_hbm.at[didx[b]], arows[b], gsem[b]).wait()

    def issue_scatter(b):
        pltpu.async_copy(rows[b], out_sh.at[dsc[b]], ssem[b], add=True)

    def wait_scatter(b):
        pltpu.make_async_copy(rows[b], out_sh.at[dsc[b]], ssem[b]).wait()

    def compute(b):
        rb = rows[b]
        ab = arows[b]
        for j in range(CHUNK // 16):
            ridx = lane + j * 16
            a = plsc.load_gather(rb, [ridx, c128]) + plsc.load_gather(ab, [ridx, c0])
            e = jnp.where(a >= 0.0, a, 0.2 * a)
            p = jnp.exp(e)
            p_v[pl.ds(j * 16, 16)] = p
            plsc.store_scatter(rb, [ridx, c128], p)

        @pl.loop(0, CHUNK, step=8)
        def _scale(r0):
            for rr in range(8):
                pb = plsc.load_gather(p_v, [jnp.full((16,), r0 + rr, jnp.int32)])
                for k in range(DIM_H // 16):
                    rb[r0 + rr, pl.ds(k * 16, 16)] = rb[r0 + rr, pl.ds(k * 16, 16)] * pb

    def half(i, b):
        b1 = 1 - b
        wait_idx(i + 1, b1)

        @pl.when(i >= 1)
        def _():
            wait_scatter(b1)

        issue_gather(b1)
        wait_gather(b)
        for k in range(CHUNK // 16):
            dsc[b][pl.ds(k * 16, 16)] = didx[b][pl.ds(k * 16, 16)]

        @pl.when(i + 2 < NCHUNKS)
        def _():
            issue_idx(i + 2, b)

        compute(b)
        issue_scatter(b)

    issue_idx(0, 0, sync=True)
    issue_gather(0)
    issue_idx(1, 1)

    @pl.loop(0, NCHUNKS - 1, step=2)
    def _main(i):
        half(i, 0)
        half(i + 1, 1)

    wait_scatter(1)
    wait_gather(0)
    for k in range(CHUNK // 16):
        dsc0[pl.ds(k * 16, 16)] = didx0[pl.ds(k * 16, 16)]
    compute(0)
    issue_scatter(0)
    wait_scatter(0)

    plsc.subcore_barrier()
    pltpu.sync_copy(out_sh.at[pl.ds(base_row, ROWS_PER_SUB), pl.ds(0, DIM_H)],
                    feat_hbm.at[pl.ds(c * N_NODES + base_row, ROWS_PER_SUB)])
    pltpu.sync_copy(out_sh.at[pl.ds(base_row, ROWS_PER_SUB), pl.ds(DIM_H, 16)],
                    den_hbm.at[pl.ds(c * N_NODES + base_row, ROWS_PER_SUB)])


def _sc_gat(xe, ad2, src_i, dst_i):
    mesh = plsc.VectorSubcoreMesh(core_axis_name="c", subcore_axis_name="s")
    cp = pltpu.CompilerParams(use_tc_tiling_on_sc=False)
    if "needs_layout_passes" in pltpu.CompilerParams.__dataclass_fields__:
        cp = dataclasses.replace(cp, needs_layout_passes=False)
    kern = pl.kernel(
        _sc_body,
        out_type=[
            jax.ShapeDtypeStruct((2 * N_NODES, DIM_H), jnp.float32),
            jax.ShapeDtypeStruct((2 * N_NODES, 16), jnp.float32),
        ],
        mesh=mesh,
        scratch_types=[
            pltpu.VMEM((CHUNK, DIM_EXT), jnp.float32),
            pltpu.VMEM((CHUNK, DIM_EXT), jnp.float32),
            pltpu.VMEM((CHUNK, 16), jnp.float32),
            pltpu.VMEM((CHUNK, 16), jnp.float32),
            pltpu.VMEM((CHUNK,), jnp.int32),
            pltpu.VMEM((CHUNK,), jnp.int32),
            pltpu.VMEM((CHUNK,), jnp.int32),
            pltpu.VMEM((CHUNK,), jnp.int32),
            pltpu.VMEM((CHUNK,), jnp.int32),
            pltpu.VMEM((CHUNK,), jnp.int32),
            pltpu.VMEM((CHUNK,), jnp.float32),
            pltpu.VMEM_SHARED((N_NODES, DIM_EXT), jnp.float32),
            pltpu.SemaphoreType.DMA,
            pltpu.SemaphoreType.DMA,
            pltpu.SemaphoreType.DMA,
            pltpu.SemaphoreType.DMA,
            pltpu.SemaphoreType.DMA,
            pltpu.SemaphoreType.DMA,
        ],
        compiler_params=cp,
    )
    return kern(xe, ad2, src_i, dst_i)


def _post_body(f0_ref, f1_ref, d0_ref, d1_ref, bias_ref, wl_ref, bl_ref,
               o_ref):
    acc = f0_ref[...] + f1_ref[...]
    den = jnp.sum(d0_ref[...] + d1_ref[...], axis=1, keepdims=True)
    h = jnp.maximum(acc / (den + 1e-16) + bias_ref[...], 0.0)
    o_ref[...] = _dot(h, wl_ref[...]) + bl_ref[...]


def _tc_post(feat, den, bias, W_lin, b_lin):
    blk = 2000
    grid = (N_NODES // blk,)
    half_off = N_NODES // blk
    return pl.pallas_call(
        _post_body,
        grid=grid,
        in_specs=[
            pl.BlockSpec((blk, DIM_H), lambda i: (i, 0)),
            pl.BlockSpec((blk, DIM_H), lambda i, o=half_off: (i + o, 0)),
            pl.BlockSpec((blk, 16), lambda i: (i, 0)),
            pl.BlockSpec((blk, 16), lambda i, o=half_off: (i + o, 0)),
            pl.BlockSpec((1, DIM_H), lambda i: (0, 0)),
            pl.BlockSpec((DIM_H, DIM_H), lambda i: (0, 0)),
            pl.BlockSpec((1, DIM_H), lambda i: (0, 0)),
        ],
        out_specs=pl.BlockSpec((blk, DIM_H), lambda i: (i, 0)),
        out_shape=jax.ShapeDtypeStruct((N_NODES, DIM_H), jnp.float32),
    )(feat, feat, den, den, bias, W_lin, b_lin)


def kernel(x, edge_index, W_src, W_dst, att_src, att_dst, bias_conv,
           W_lin, b_lin):
    src_i = edge_index[0].astype(jnp.int32)
    dst_i = edge_index[1].astype(jnp.int32)

    atts16 = jnp.zeros((DIM_H, 16), jnp.float32).at[:, 0].set(att_src)
    attd16 = jnp.zeros((DIM_H, 16), jnp.float32).at[:, 0].set(att_dst)
    xe, ad2 = _tc_pre(x, W_src, W_dst, atts16, attd16)

    feat, den = _sc_gat(xe, ad2, src_i, dst_i)

    return _tc_post(feat, den, bias_conv.reshape(1, DIM_H),
                    W_lin, b_lin.reshape(1, DIM_H))


# speedup vs baseline: 1.4308x; 1.0411x over previous
import dataclasses
import functools

import jax
import jax.numpy as jnp
from jax import lax
from jax.experimental import pallas as pl
from jax.experimental.pallas import tpu as pltpu
from jax.experimental.pallas import tpu_sc as plsc

N_NODES = 10000
D_FEAT = 128
DIM_H = 128
DIM_EXT = 144
N_EDGES = 320000
CHUNK = 80
EDGES_PER_TILE = N_EDGES // 32
NCHUNKS = EDGES_PER_TILE // CHUNK
ROWS_PER_SUB = N_NODES // 16


def _dot(a, b):
    return lax.dot_general(a, b, (((1,), (0,)), ((), ())),
                           preferred_element_type=jnp.float32)


def _pre_body(x_ref, ws_ref, wd_ref, atts_ref, attd_ref, xe_ref, ad_ref):
    x = x_ref[...]
    xs = _dot(x, ws_ref[...])
    asrc16 = _dot(xs, atts_ref[...])
    xe_ref[...] = jnp.concatenate([xs, asrc16], axis=1)
    xd = _dot(x, wd_ref[...])
    ad_ref[...] = _dot(xd, attd_ref[...])


def _tc_pre(x, W_src, W_dst, atts16, attd16):
    blk = 2000
    grid = (N_NODES // blk,)
    return pl.pallas_call(
        _pre_body,
        grid=grid,
        in_specs=[
            pl.BlockSpec((blk, D_FEAT), lambda i: (i, 0)),
            pl.BlockSpec((D_FEAT, DIM_H), lambda i: (0, 0)),
            pl.BlockSpec((D_FEAT, DIM_H), lambda i: (0, 0)),
            pl.BlockSpec((DIM_H, 16), lambda i: (0, 0)),
            pl.BlockSpec((DIM_H, 16), lambda i: (0, 0)),
        ],
        out_specs=[
            pl.BlockSpec((blk, DIM_EXT), lambda i: (i, 0)),
            pl.BlockSpec((blk, 16), lambda i: (i, 0)),
        ],
        out_shape=[
            jax.ShapeDtypeStruct((N_NODES, DIM_EXT), jnp.float32),
            jax.ShapeDtypeStruct((N_NODES, 16), jnp.float32),
        ],
    )(x, W_src, W_dst, atts16, attd16)


def _sc_body(xe_hbm, ad2_hbm, ei_hbm, feat_hbm, den_hbm,
             rows0, rows1, arows0, arows1, sidx0, sidx1, didx0, didx1,
             dsc0, dsc1, p_v, out_sh,
             gsem0, gsem1, ssem0, ssem1, isem0, isem1):
    c = lax.axis_index("c")
    s = lax.axis_index("s")
    wid = c * 16 + s
    edge0 = wid * EDGES_PER_TILE

    rows = (rows0, rows1)
    arows = (arows0, arows1)
    sidx = (sidx0, sidx1)
    didx = (didx0, didx1)
    dsc = (dsc0, dsc1)
    gsem = (gsem0, gsem1)
    ssem = (ssem0, ssem1)
    isem = (isem0, isem1)

    zf = jnp.zeros((16,), jnp.float32)
    lane = lax.iota(jnp.int32, 16)
    c128 = jnp.full((16,), DIM_H, jnp.int32)
    c0 = jnp.zeros((16,), jnp.int32)

    @pl.loop(0, CHUNK)
    def _zero(r):
        for k in range(DIM_EXT // 16):
            rows0[r, pl.ds(k * 16, 16)] = zf

    base_row = s * ROWS_PER_SUB
    for t in range(7):
        pltpu.sync_copy(rows0, out_sh.at[pl.ds(base_row + t * CHUNK, CHUNK)])
    pltpu.sync_copy(rows0.at[pl.ds(0, 65)],
                    out_sh.at[pl.ds(base_row + 7 * CHUNK, 65)])
    plsc.subcore_barrier()

    def issue_idx(i, b, sync=False):
        off = edge0 + i * CHUNK
        if sync:
            pltpu.sync_copy(ei_hbm.at[0, pl.ds(off, CHUNK)], sidx[b])
            pltpu.sync_copy(ei_hbm.at[1, pl.ds(off, CHUNK)], didx[b])
        else:
            pltpu.async_copy(ei_hbm.at[0, pl.ds(off, CHUNK)], sidx[b], isem[b])
            pltpu.async_copy(ei_hbm.at[1, pl.ds(off, CHUNK)], didx[b], isem[b])

    def wait_idx(i, b):
        off = edge0 + i * CHUNK
        pltpu.make_async_copy(ei_hbm.at[0, pl.ds(off, CHUNK)], sidx[b], isem[b]).wait()
        pltpu.make_async_copy(ei_hbm.at[1, pl.ds(off, CHUNK)], didx[b], isem[b]).wait()

    def issue_gather(b):
        pltpu.async_copy(xe_hbm.at[sidx[b]], rows[b], gsem[b])
        pltpu.async_copy(ad2_hbm.at[didx[b]], arows[b], gsem[b])

    def wait_gather(b):
        pltpu.make_async_copy(xe_hbm.at[sidx[b]], rows[b], gsem[b]).wait()
        pltpu.make_async_copy(ad2_hbm.at[didx[b]], arows[b], gsem[b]).wait()

    def issue_scatter(b):
        pltpu.async_copy(rows[b], out_sh.at[dsc[b]], ssem[b], add=True)

    def wait_scatter(b):
        pltpu.make_async_copy(rows[b], out_sh.at[dsc[b]], ssem[b]).wait()

    def compute(b):
        rb = rows[b]
        ab = arows[b]
        for j in range(CHUNK // 16):
            ridx = lane + j * 16
            a = plsc.load_gather(rb, [ridx, c128]) + plsc.load_gather(ab, [ridx, c0])
            e = jnp.where(a >= 0.0, a, 0.2 * a)
            p = jnp.exp(e)
            p_v[pl.ds(j * 16, 16)] = p
            plsc.store_scatter(rb, [ridx, c128], p)

        @pl.loop(0, CHUNK, step=8)
        def _scale(r0):
            for rr in range(8):
                pb = plsc.load_gather(p_v, [jnp.full((16,), r0 + rr, jnp.int32)])
                for k in range(DIM_H // 16):
                    rb[r0 + rr, pl.ds(k * 16, 16)] = rb[r0 + rr, pl.ds(k * 16, 16)] * pb

    def half(i, b):
        b1 = 1 - b
        wait_idx(i + 1, b1)

        @pl.when(i >= 1)
        def _():
            wait_scatter(b1)

        issue_gather(b1)
        wait_gather(b)
        for k in range(CHUNK // 16):
            dsc[b][pl.ds(k * 16, 16)] = didx[b][pl.ds(k * 16, 16)]

        @pl.when(i + 2 < NCHUNKS)
        def _():
            issue_idx(i + 2, b)

        compute(b)
        issue_scatter(b)

    issue_idx(0, 0, sync=True)
    issue_gather(0)
    issue_idx(1, 1)

    @pl.loop(0, NCHUNKS - 1, step=2)
    def _main(i):
        half(i, 0)
        half(i + 1, 1)

    wait_scatter(1)
    wait_gather(0)
    for k in range(CHUNK // 16):
        dsc0[pl.ds(k * 16, 16)] = didx0[pl.ds(k * 16, 16)]
    compute(0)
    issue_scatter(0)
    wait_scatter(0)

    plsc.subcore_barrier()
    pltpu.sync_copy(out_sh.at[pl.ds(base_row, ROWS_PER_SUB), pl.ds(0, DIM_H)],
                    feat_hbm.at[pl.ds(c * N_NODES + base_row, ROWS_PER_SUB)])
    pltpu.sync_copy(out_sh.at[pl.ds(base_row, ROWS_PER_SUB), pl.ds(DIM_H, 16)],
                    den_hbm.at[pl.ds(c * N_NODES + base_row, ROWS_PER_SUB)])


def _sc_gat(xe, ad2, ei):
    mesh = plsc.VectorSubcoreMesh(core_axis_name="c", subcore_axis_name="s")
    cp = pltpu.CompilerParams(use_tc_tiling_on_sc=False)
    if "needs_layout_passes" in pltpu.CompilerParams.__dataclass_fields__:
        cp = dataclasses.replace(cp, needs_layout_passes=False)
    kern = pl.kernel(
        _sc_body,
        out_type=[
            jax.ShapeDtypeStruct((2 * N_NODES, DIM_H), jnp.float32),
            jax.ShapeDtypeStruct((2 * N_NODES, 16), jnp.float32),
        ],
        mesh=mesh,
        scratch_types=[
            pltpu.VMEM((CHUNK, DIM_EXT), jnp.float32),
            pltpu.VMEM((CHUNK, DIM_EXT), jnp.float32),
            pltpu.VMEM((CHUNK, 16), jnp.float32),
            pltpu.VMEM((CHUNK, 16), jnp.float32),
            pltpu.VMEM((CHUNK,), jnp.int32),
            pltpu.VMEM((CHUNK,), jnp.int32),
            pltpu.VMEM((CHUNK,), jnp.int32),
            pltpu.VMEM((CHUNK,), jnp.int32),
            pltpu.VMEM((CHUNK,), jnp.int32),
            pltpu.VMEM((CHUNK,), jnp.int32),
            pltpu.VMEM((CHUNK,), jnp.float32),
            pltpu.VMEM_SHARED((N_NODES, DIM_EXT), jnp.float32),
            pltpu.SemaphoreType.DMA,
            pltpu.SemaphoreType.DMA,
            pltpu.SemaphoreType.DMA,
            pltpu.SemaphoreType.DMA,
            pltpu.SemaphoreType.DMA,
            pltpu.SemaphoreType.DMA,
        ],
        compiler_params=cp,
    )
    return kern(xe, ad2, ei)


def _post_body(f0_ref, f1_ref, d0_ref, d1_ref, bias_ref, wl_ref, bl_ref,
               o_ref):
    acc = f0_ref[...] + f1_ref[...]
    den = jnp.sum(d0_ref[...] + d1_ref[...], axis=1, keepdims=True)
    h = jnp.maximum(acc / (den + 1e-16) + bias_ref[...], 0.0)
    o_ref[...] = _dot(h, wl_ref[...]) + bl_ref[...]


def _tc_post(feat, den, bias, W_lin, b_lin):
    blk = 2000
    grid = (N_NODES // blk,)
    half_off = N_NODES // blk
    return pl.pallas_call(
        _post_body,
        grid=grid,
        in_specs=[
            pl.BlockSpec((blk, DIM_H), lambda i: (i, 0)),
            pl.BlockSpec((blk, DIM_H), lambda i, o=half_off: (i + o, 0)),
            pl.BlockSpec((blk, 16), lambda i: (i, 0)),
            pl.BlockSpec((blk, 16), lambda i, o=half_off: (i + o, 0)),
            pl.BlockSpec((1, DIM_H), lambda i: (0, 0)),
            pl.BlockSpec((DIM_H, DIM_H), lambda i: (0, 0)),
            pl.BlockSpec((1, DIM_H), lambda i: (0, 0)),
        ],
        out_specs=pl.BlockSpec((blk, DIM_H), lambda i: (i, 0)),
        out_shape=jax.ShapeDtypeStruct((N_NODES, DIM_H), jnp.float32),
    )(feat, feat, den, den, bias, W_lin, b_lin)


def kernel(x, edge_index, W_src, W_dst, att_src, att_dst, bias_conv,
           W_lin, b_lin):
    ei = edge_index.astype(jnp.int32)

    atts16 = jnp.zeros((DIM_H, 16), jnp.float32).at[:, 0].set(att_src)
    attd16 = jnp.zeros((DIM_H, 16), jnp.float32).at[:, 0].set(att_dst)
    xe, ad2 = _tc_pre(x, W_src, W_dst, atts16, attd16)

    feat, den = _sc_gat(xe, ad2, ei)

    return _tc_post(feat, den, bias_conv.reshape(1, DIM_H),
                    W_lin, b_lin.reshape(1, DIM_H))
